# baseline (speedup 1.0000x reference)
import jax
import jax.numpy as jnp
from jax.experimental import pallas as pl
from jax.experimental.pallas import tpu as pltpu

_NUM_PATCHES = 1024
_DIM = 768
_BP = 512


def _add_kernel(x_ref, t_ref, o_ref):
    o_ref[...] = x_ref[...] + t_ref[...][None]


def kernel(x, table):
    batch, num_patches, dim = x.shape
    grid = (batch, num_patches // _BP)
    return pl.pallas_call(
        _add_kernel,
        grid=grid,
        in_specs=[
            pl.BlockSpec((1, _BP, dim), lambda b, p: (b, p, 0)),
            pl.BlockSpec((_BP, dim), lambda b, p: (p, 0)),
        ],
        out_specs=pl.BlockSpec((1, _BP, dim), lambda b, p: (b, p, 0)),
        out_shape=jax.ShapeDtypeStruct(x.shape, x.dtype),
        compiler_params=pltpu.CompilerParams(
            dimension_semantics=("parallel", "arbitrary"),
        ),
    )(x, table)


# speedup vs baseline: 1.5727x; 1.5727x over previous
import jax
import jax.numpy as jnp
from jax.experimental import pallas as pl
from jax.experimental.pallas import tpu as pltpu

_BB = 2


def _add_kernel(x_ref, t_ref, o_ref):
    o_ref[...] = x_ref[...] + t_ref[...][None]


def kernel(x, table):
    batch, num_patches, dim = x.shape
    grid = (batch // _BB,)
    return pl.pallas_call(
        _add_kernel,
        grid=grid,
        in_specs=[
            pl.BlockSpec((_BB, num_patches, dim), lambda b: (b, 0, 0)),
            pl.BlockSpec((num_patches, dim), lambda b: (0, 0)),
        ],
        out_specs=pl.BlockSpec((_BB, num_patches, dim), lambda b: (b, 0, 0)),
        out_shape=jax.ShapeDtypeStruct(x.shape, x.dtype),
        compiler_params=pltpu.CompilerParams(
            dimension_semantics=("parallel",),
        ),
    )(x, table)


# speedup vs baseline: 1.5892x; 1.0105x over previous
import jax
import jax.numpy as jnp
from jax.experimental import pallas as pl
from jax.experimental.pallas import tpu as pltpu

_BB = 4


def _add_kernel(x_ref, t_ref, o_ref):
    o_ref[...] = x_ref[...] + t_ref[...][None]


def kernel(x, table):
    batch, num_patches, dim = x.shape
    grid = (batch // _BB,)
    return pl.pallas_call(
        _add_kernel,
        grid=grid,
        in_specs=[
            pl.BlockSpec((_BB, num_patches, dim), lambda b: (b, 0, 0)),
            pl.BlockSpec((num_patches, dim), lambda b: (0, 0)),
        ],
        out_specs=pl.BlockSpec((_BB, num_patches, dim), lambda b: (b, 0, 0)),
        out_shape=jax.ShapeDtypeStruct(x.shape, x.dtype),
        compiler_params=pltpu.CompilerParams(
            dimension_semantics=("parallel",),
        ),
    )(x, table)
